# Initial kernel scaffold; baseline (speedup 1.0000x reference)
import jax
import jax.numpy as jnp
from jax.experimental import pallas as pl


def kernel(h, edge_index):
    raise NotImplementedError("write your pallas kernel here")


# speedup vs baseline: 4.4296x; 4.4296x over previous
import functools

import jax
import jax.numpy as jnp
from jax import lax
from jax.experimental import pallas as pl
from jax.experimental.pallas import tpu as pltpu
from jax.experimental.pallas import tpu_sc as plsc

_NC = 2
_NS = 16
_NW = _NC * _NS
_L = 16
_C = 80


def _sc_edge_dot(h, src, dst):
    V, D = h.shape
    E = src.shape[0]
    epw = E // _NW
    n_chunks = epw // _C
    kd = D // _L

    mesh = plsc.VectorSubcoreMesh(core_axis_name="c", subcore_axis_name="s")

    @functools.partial(
        pl.kernel,
        mesh=mesh,
        out_type=jax.ShapeDtypeStruct((E,), jnp.float32),
        scratch_types=[
            pltpu.VMEM((n_chunks, _C), jnp.int32),
            pltpu.VMEM((n_chunks, _C), jnp.int32),
            pltpu.VMEM((_C, D), jnp.float32),
            pltpu.VMEM((_C, D), jnp.float32),
            pltpu.VMEM((_L * _L,), jnp.float32),
            pltpu.VMEM((epw,), jnp.float32),
            pltpu.SemaphoreType.DMA,
            pltpu.SemaphoreType.DMA,
        ],
        compiler_params=pltpu.CompilerParams(needs_layout_passes=False),
    )
    def body(h_hbm, src_hbm, dst_hbm, out_hbm,
             sidx, didx, srows, drows, tsc, obuf, sem_s, sem_d):
        wid = lax.axis_index("s") * _NC + lax.axis_index("c")
        pltpu.sync_copy(src_hbm.at[wid], sidx)
        pltpu.sync_copy(dst_hbm.at[wid], didx)

        lane_ids = lax.iota(jnp.int32, _L)

        def chunk_body(t, carry):
            cp_s = pltpu.async_copy(h_hbm.at[sidx.at[t]], srows, sem_s)
            cp_d = pltpu.async_copy(h_hbm.at[didx.at[t]], drows, sem_d)
            cp_s.wait()
            cp_d.wait()

            def group_body(g, carry2):
                row0 = g * _L
                for e in range(_L):
                    row = row0 + e
                    acc = srows[row, pl.ds(0, _L)] * drows[row, pl.ds(0, _L)]
                    for k in range(1, kd):
                        acc = acc + (srows[row, pl.ds(k * _L, _L)]
                                     * drows[row, pl.ds(k * _L, _L)])
                    tsc[pl.ds(e * _L, _L)] = acc
                tot = plsc.load_gather(tsc, [lane_ids * _L])
                for j in range(1, _L):
                    tot = tot + plsc.load_gather(tsc, [lane_ids * _L + j])
                off = pl.multiple_of(t * _C + row0, _L)
                obuf[pl.ds(off, _L)] = tot
                return carry2

            lax.fori_loop(0, _C // _L, group_body, 0, unroll=False)
            return carry

        lax.fori_loop(0, n_chunks, chunk_body, 0, unroll=False)
        base = pl.multiple_of(wid * epw, 8)
        pltpu.sync_copy(obuf, out_hbm.at[pl.ds(base, epw)])

    return body(h, src.reshape(_NW, n_chunks, _C), dst.reshape(_NW, n_chunks, _C))


def kernel(h, edge_index):
    E = edge_index.shape[1]
    src = edge_index[0].astype(jnp.int32)
    dst = edge_index[1].astype(jnp.int32)
    scores = _sc_edge_dot(h, src, dst)
    return scores.reshape(E, 1)


# speedup vs baseline: 7.5815x; 1.7116x over previous
import functools

import jax
import jax.numpy as jnp
from jax import lax
from jax.experimental import pallas as pl
from jax.experimental.pallas import tpu as pltpu
from jax.experimental.pallas import tpu_sc as plsc

_NC = 2
_NS = 16
_NW = _NC * _NS
_L = 16
_C = 80


def _sc_edge_dot(h, src, dst):
    V, D = h.shape
    E = src.shape[0]
    epw = E // _NW
    n_chunks = epw // _C
    kd = D // _L

    mesh = plsc.VectorSubcoreMesh(core_axis_name="c", subcore_axis_name="s")

    @functools.partial(
        pl.kernel,
        mesh=mesh,
        out_type=jax.ShapeDtypeStruct((E,), jnp.float32),
        scratch_types=[
            pltpu.VMEM((n_chunks, _C), jnp.int32),
            pltpu.VMEM((n_chunks, _C), jnp.int32),
            pltpu.VMEM((_C, D), jnp.float32),
            pltpu.VMEM((_C, D), jnp.float32),
            pltpu.VMEM((_C, D), jnp.float32),
            pltpu.VMEM((_C, D), jnp.float32),
            pltpu.VMEM((_L * _L,), jnp.float32),
            pltpu.VMEM((epw,), jnp.float32),
            pltpu.SemaphoreType.DMA,
            pltpu.SemaphoreType.DMA,
            pltpu.SemaphoreType.DMA,
            pltpu.SemaphoreType.DMA,
        ],
        compiler_params=pltpu.CompilerParams(needs_layout_passes=False),
    )
    def body(h_hbm, src_hbm, dst_hbm, out_hbm,
             sidx, didx, srows0, srows1, drows0, drows1, tsc, obuf,
             sem_s0, sem_s1, sem_d0, sem_d1):
        wid = lax.axis_index("s") * _NC + lax.axis_index("c")
        pltpu.sync_copy(src_hbm.at[wid], sidx)
        pltpu.sync_copy(dst_hbm.at[wid], didx)

        lane_ids = lax.iota(jnp.int32, _L)

        pltpu.async_copy(h_hbm.at[sidx.at[0]], srows0, sem_s0)
        pltpu.async_copy(h_hbm.at[didx.at[0]], drows0, sem_d0)
        pltpu.async_copy(h_hbm.at[sidx.at[1]], srows1, sem_s1)
        pltpu.async_copy(h_hbm.at[didx.at[1]], drows1, sem_d1)

        def process(t, srows, drows, sem_s, sem_d):
            pltpu.make_async_copy(h_hbm.at[sidx.at[t]], srows, sem_s).wait()
            pltpu.make_async_copy(h_hbm.at[didx.at[t]], drows, sem_d).wait()

            def group_body(g, carry2):
                row0 = g * _L
                for e in range(_L):
                    row = row0 + e
                    acc = srows[row, pl.ds(0, _L)] * drows[row, pl.ds(0, _L)]
                    for k in range(1, kd):
                        acc = acc + (srows[row, pl.ds(k * _L, _L)]
                                     * drows[row, pl.ds(k * _L, _L)])
                    tsc[pl.ds(e * _L, _L)] = acc
                tot = plsc.load_gather(tsc, [lane_ids * _L])
                for j in range(1, _L):
                    tot = tot + plsc.load_gather(tsc, [lane_ids * _L + j])
                off = pl.multiple_of(t * _C + row0, _L)
                obuf[pl.ds(off, _L)] = tot
                return carry2

            lax.fori_loop(0, _C // _L, group_body, 0, unroll=False)

            @pl.when(t + 2 < n_chunks)
            def _():
                pltpu.async_copy(h_hbm.at[sidx.at[t + 2]], srows, sem_s)
                pltpu.async_copy(h_hbm.at[didx.at[t + 2]], drows, sem_d)

        def chunk_body(t, carry):
            @pl.when(t % 2 == 0)
            def _():
                process(t, srows0, drows0, sem_s0, sem_d0)

            @pl.when(t % 2 == 1)
            def _():
                process(t, srows1, drows1, sem_s1, sem_d1)

            return carry

        lax.fori_loop(0, n_chunks, chunk_body, 0, unroll=False)
        base = pl.multiple_of(wid * epw, 8)
        pltpu.sync_copy(obuf, out_hbm.at[pl.ds(base, epw)])

    return body(h, src.reshape(_NW, n_chunks, _C), dst.reshape(_NW, n_chunks, _C))


def kernel(h, edge_index):
    E = edge_index.shape[1]
    src = edge_index[0].astype(jnp.int32)
    dst = edge_index[1].astype(jnp.int32)
    scores = _sc_edge_dot(h, src, dst)
    return scores.reshape(E, 1)
